# Initial kernel scaffold; baseline (speedup 1.0000x reference)
import jax
import jax.numpy as jnp
from jax.experimental import pallas as pl


def kernel(nfeat, efeat, edge_index):
    raise NotImplementedError("write your pallas kernel here")


# speedup vs baseline: 2.7058x; 2.7058x over previous
import functools

import jax
import jax.numpy as jnp
from jax import lax
from jax.experimental import pallas as pl
from jax.experimental.pallas import tpu as pltpu
from jax.experimental.pallas import tpu_sc as plsc

_LANES = 16
_IDXW = 128
E_PAD = 2560 * _IDXW
N_PAD = 16 * 626
_IDX_ROWS = E_PAD // _IDXW
_P1_ROWS_PER_TILE = _IDX_ROWS // 16
_P2_ROWS_PER_TILE = _IDX_ROWS // 32
_P1_CHUNK = 32
_P2_CHUNK = 16


@functools.partial(
    pl.kernel,
    out_type=jax.ShapeDtypeStruct((E_PAD, _LANES), jnp.float32),
    mesh=plsc.VectorSubcoreMesh(
        core_axis_name="c", subcore_axis_name="s", num_cores=2, num_subcores=16
    ),
    scratch_types=[
        pltpu.VMEM_SHARED((N_PAD, _LANES), jnp.float32),
        pltpu.VMEM((4096, _LANES), jnp.float32),
        pltpu.VMEM((_P1_CHUNK, _IDXW), jnp.int32),
    ],
    compiler_params=pltpu.CompilerParams(use_tc_tiling_on_sc=False),
)
def _sc_dmpnn(efeat_hbm, dst_hbm, src_hbm, out_hbm, neigh, buf, idx_v):
    c = lax.axis_index("c")
    s = lax.axis_index("s")

    zrows = N_PAD // 16

    def _zero(i, carry):
        buf[i] = jnp.zeros((_LANES,), jnp.float32)
        return carry

    lax.fori_loop(0, zrows, _zero, 0)
    pltpu.sync_copy(buf.at[pl.ds(0, zrows)], neigh.at[pl.ds(s * zrows, zrows)])
    plsc.subcore_barrier()

    for chunk in range(_P1_ROWS_PER_TILE // _P1_CHUNK):
        rbase = s * _P1_ROWS_PER_TILE + chunk * _P1_CHUNK
        pltpu.sync_copy(dst_hbm.at[pl.ds(rbase, _P1_CHUNK)], idx_v)
        pltpu.sync_copy(
            efeat_hbm.at[pl.ds(rbase * _IDXW, _P1_CHUNK * _IDXW)], buf
        )
        for j in range(_P1_CHUNK):
            pltpu.sync_copy(
                buf.at[pl.ds(j * _IDXW, _IDXW)], neigh.at[idx_v.at[j]], add=True
            )
    plsc.subcore_barrier()

    wid = c * 16 + s
    nrows = _P2_CHUNK * _IDXW

    for chunk in range(_P2_ROWS_PER_TILE // _P2_CHUNK):
        rbase = wid * _P2_ROWS_PER_TILE + chunk * _P2_CHUNK
        ebase = rbase * _IDXW
        pltpu.sync_copy(
            src_hbm.at[pl.ds(rbase, _P2_CHUNK)], idx_v.at[pl.ds(0, _P2_CHUNK)]
        )
        for j in range(_P2_CHUNK):
            pltpu.sync_copy(
                neigh.at[idx_v.at[j]], buf.at[pl.ds(j * _IDXW, _IDXW)]
            )
        pltpu.sync_copy(
            efeat_hbm.at[pl.ds(ebase, nrows)], buf.at[pl.ds(nrows, nrows)]
        )

        def _sub(p, carry):
            a = buf[nrows + 2 * p]
            b = buf[nrows + 2 * p + 1]
            buf[2 * p] = buf[2 * p] - b
            buf[2 * p + 1] = buf[2 * p + 1] - a
            return carry

        lax.fori_loop(0, nrows // 2, _sub, 0)
        pltpu.sync_copy(buf.at[pl.ds(0, nrows)], out_hbm.at[pl.ds(ebase, nrows)])


def kernel(nfeat, efeat, edge_index):
    n_nodes = nfeat.shape[0]
    e = efeat.shape[0]
    pad = E_PAD - e
    src = edge_index[0]
    dst = edge_index[1]
    dst_p = jnp.concatenate(
        [dst, jnp.full((pad,), n_nodes, jnp.int32)]
    ).reshape(_IDX_ROWS, _IDXW)
    src_p = jnp.concatenate([src, jnp.zeros((pad,), jnp.int32)]).reshape(
        _IDX_ROWS, _IDXW
    )
    efeat_p = jnp.concatenate(
        [efeat, jnp.zeros((pad, efeat.shape[1]), jnp.float32)]
    )
    out = _sc_dmpnn(efeat_p, dst_p, src_p)
    return out[:e]


# speedup vs baseline: 4.2418x; 1.5677x over previous
import functools

import jax
import jax.numpy as jnp
from jax import lax
from jax.experimental import pallas as pl
from jax.experimental.pallas import tpu as pltpu
from jax.experimental.pallas import tpu_sc as plsc

_LANES = 16
_IDXW = 128
_E = 320000
_N = 10000
_IDX_ROWS = _E // _IDXW
N_PAD = 16 * 626
_P1_BASE = _IDX_ROWS // 16
_P1_REM = _IDX_ROWS - 16 * _P1_BASE
_P2_BASE = _IDX_ROWS // 32
_P2_REM = _IDX_ROWS - 32 * _P2_BASE
_P1_CHUNK = 26
_P2_CHUNK = 26


@functools.partial(
    pl.kernel,
    out_type=jax.ShapeDtypeStruct((_E, _LANES), jnp.float32),
    mesh=plsc.VectorSubcoreMesh(
        core_axis_name="c", subcore_axis_name="s", num_cores=2, num_subcores=16
    ),
    scratch_types=[
        pltpu.VMEM_SHARED((N_PAD, _LANES), jnp.float32),
        pltpu.VMEM((2 * _P1_CHUNK * _IDXW, _LANES), jnp.float32),
        pltpu.VMEM((_P1_CHUNK, _IDXW), jnp.int32),
    ],
    compiler_params=pltpu.CompilerParams(use_tc_tiling_on_sc=False),
)
def _sc_dmpnn(efeat_hbm, eidx_hbm, out_hbm, neigh, buf, idx_v):
    c = lax.axis_index("c")
    s = lax.axis_index("s")
    dst_hbm = eidx_hbm.at[1]
    src_hbm = eidx_hbm.at[0]

    zrows = N_PAD // 16

    def _zero(i, carry):
        buf[i] = jnp.zeros((_LANES,), jnp.float32)
        return carry

    lax.fori_loop(0, zrows, _zero, 0)
    pltpu.sync_copy(buf.at[pl.ds(0, zrows)], neigh.at[pl.ds(s * zrows, zrows)])
    plsc.subcore_barrier()

    for chunk in range(_P1_BASE // _P1_CHUNK):
        rbase = s * _P1_BASE + chunk * _P1_CHUNK
        pltpu.sync_copy(dst_hbm.at[pl.ds(rbase, _P1_CHUNK)], idx_v.at[pl.ds(0, _P1_CHUNK)])
        pltpu.sync_copy(
            efeat_hbm.at[pl.ds(rbase * _IDXW, _P1_CHUNK * _IDXW)],
            buf.at[pl.ds(0, _P1_CHUNK * _IDXW)],
        )
        for j in range(_P1_CHUNK):
            pltpu.sync_copy(
                buf.at[pl.ds(j * _IDXW, _IDXW)], neigh.at[idx_v.at[j]], add=True
            )

    @pl.when(s < _P1_REM)
    def _p1_rem():
        row = 16 * _P1_BASE + s
        pltpu.sync_copy(dst_hbm.at[pl.ds(row, 1)], idx_v.at[pl.ds(0, 1)])
        pltpu.sync_copy(
            efeat_hbm.at[pl.ds(row * _IDXW, _IDXW)], buf.at[pl.ds(0, _IDXW)]
        )
        pltpu.sync_copy(buf.at[pl.ds(0, _IDXW)], neigh.at[idx_v.at[0]], add=True)

    plsc.subcore_barrier()

    wid = c * 16 + s
    nrows = _P2_CHUNK * _IDXW

    def _p2_chunk(rbase, n_idx_rows):
        n_e = n_idx_rows * _IDXW
        ebase = rbase * _IDXW
        pltpu.sync_copy(
            src_hbm.at[pl.ds(rbase, n_idx_rows)], idx_v.at[pl.ds(0, n_idx_rows)]
        )
        for j in range(n_idx_rows):
            pltpu.sync_copy(
                neigh.at[idx_v.at[j]], buf.at[pl.ds(j * _IDXW, _IDXW)]
            )
        pltpu.sync_copy(
            efeat_hbm.at[pl.ds(ebase, n_e)], buf.at[pl.ds(nrows, n_e)]
        )

        def _sub(p, carry):
            a = buf[nrows + 2 * p]
            b = buf[nrows + 2 * p + 1]
            buf[2 * p] = buf[2 * p] - b
            buf[2 * p + 1] = buf[2 * p + 1] - a
            return carry

        lax.fori_loop(0, n_e // 2, _sub, 0)
        pltpu.sync_copy(buf.at[pl.ds(0, n_e)], out_hbm.at[pl.ds(ebase, n_e)])

    for chunk in range(_P2_BASE // _P2_CHUNK):
        _p2_chunk(wid * _P2_BASE + chunk * _P2_CHUNK, _P2_CHUNK)

    @pl.when(wid < _P2_REM)
    def _p2_rem():
        _p2_chunk(32 * _P2_BASE + wid, 1)


def kernel(nfeat, efeat, edge_index):
    eidx = edge_index.reshape(2, _IDX_ROWS, _IDXW)
    return _sc_dmpnn(efeat, eidx)
